# baseline (speedup 1.0000x reference)
import jax
import jax.numpy as jnp
from jax import lax
from jax.experimental import pallas as pl
from jax.experimental.pallas import tpu as pltpu
from jax.experimental.pallas import tpu_sc as plsc

_N = 10000
_E = 320000
_D = 128
_H = 128
_C = 7

_NC = 2
_NS = 16
_NW = _NC * _NS
_EB = 128
_BLKS = 80
_EPW = _BLKS * _EB
_E_PAD = _NW * _EPW
_NACC = 10240
_RPT = _NACC // _NS
_DW = 16
_RB = 2000
_GRID = _N // _RB


_sc_mesh = plsc.VectorSubcoreMesh(core_axis_name="c", subcore_axis_name="s")


def _deg_body(dst_hbm, out_hbm, ones_v, dst_v, acc):
    c = lax.axis_index("c")
    s = lax.axis_index("s")
    w = c * _NS + s

    def _fill(val):
        def row(i, carry):
            ones_v[i, :] = jnp.full((_DW,), val, jnp.float32)
            return carry
        lax.fori_loop(0, _EB, row, 0)

    _fill(0.0)
    for k in range(_RPT // _EB):
        pltpu.sync_copy(ones_v, acc.at[pl.ds(s * _RPT + k * _EB, _EB)])
    _fill(1.0)
    pltpu.sync_copy(dst_hbm.at[w], dst_v)
    plsc.subcore_barrier()

    def blk(j, carry):
        pltpu.sync_copy(ones_v, acc.at[dst_v.at[j]], add=True)
        return carry

    lax.fori_loop(0, _BLKS, blk, 0)
    plsc.subcore_barrier()
    pltpu.sync_copy(acc.at[pl.ds(s * _RPT, _RPT)],
                    out_hbm.at[c, pl.ds(s * _RPT, _RPT)])


_deg_call = pl.kernel(
    _deg_body,
    out_type=jax.ShapeDtypeStruct((_NC, _NACC, _DW), jnp.float32),
    mesh=_sc_mesh,
    scratch_types=[
        pltpu.VMEM((_EB, _DW), jnp.float32),
        pltpu.VMEM((_BLKS, _EB), jnp.int32),
        pltpu.VMEM_SHARED((_NACC, _DW), jnp.float32),
    ],
)


def _agg_body(src_hbm, dst_hbm, q_hbm, out_hbm, src_v, dst_v, rows_v,
              acc, semg, semd):
    c = lax.axis_index("c")
    s = lax.axis_index("s")
    w = c * _NS + s

    def zrow(i, carry):
        def zcol(jj, carry2):
            rows_v[i, pl.ds(pl.multiple_of(jj * 16, 16), 16)] = (
                jnp.zeros((16,), jnp.float32))
            return carry2
        return lax.fori_loop(0, _H // 16, zcol, carry)

    lax.fori_loop(0, _EB, zrow, 0)
    for k in range(_RPT // _EB):
        pltpu.sync_copy(rows_v, acc.at[pl.ds(s * _RPT + k * _EB, _EB)])
    pltpu.sync_copy(src_hbm.at[w], src_v)
    plsc.subcore_barrier()

    pltpu.async_copy(dst_hbm.at[w, 0], dst_v.at[0], semd)
    pltpu.async_copy(dst_hbm.at[w, 1], dst_v.at[1], semd)

    def blk(t, carry):
        j = t * 2
        pltpu.async_copy(q_hbm.at[src_v.at[j]], rows_v, semg).wait()
        pltpu.make_async_copy(dst_hbm.at[w, 0], dst_v.at[0], semd).wait()
        pltpu.sync_copy(rows_v, acc.at[dst_v.at[0]], add=True)
        jn = jnp.minimum(j + 2, _BLKS - 1)
        pltpu.async_copy(dst_hbm.at[w, jn], dst_v.at[0], semd)
        pltpu.async_copy(q_hbm.at[src_v.at[j + 1]], rows_v, semg).wait()
        pltpu.make_async_copy(dst_hbm.at[w, 0], dst_v.at[1], semd).wait()
        pltpu.sync_copy(rows_v, acc.at[dst_v.at[1]], add=True)
        jm = jnp.minimum(j + 3, _BLKS - 1)
        pltpu.async_copy(dst_hbm.at[w, jm], dst_v.at[1], semd)
        return carry

    lax.fori_loop(0, _BLKS // 2, blk, 0)
    pltpu.make_async_copy(dst_hbm.at[w, 0], dst_v.at[0], semd).wait()
    pltpu.make_async_copy(dst_hbm.at[w, 0], dst_v.at[1], semd).wait()
    plsc.subcore_barrier()
    pltpu.sync_copy(acc.at[pl.ds(s * _RPT, _RPT)],
                    out_hbm.at[c, pl.ds(s * _RPT, _RPT)])


_agg_call = pl.kernel(
    _agg_body,
    out_type=jax.ShapeDtypeStruct((_NC, _NACC, _H), jnp.float32),
    mesh=_sc_mesh,
    scratch_types=[
        pltpu.VMEM((_BLKS, _EB), jnp.int32),
        pltpu.VMEM((2, _EB), jnp.int32),
        pltpu.VMEM((_EB, _H), jnp.float32),
        pltpu.VMEM_SHARED((_NACC, _H), jnp.float32),
        pltpu.SemaphoreType.DMA,
        pltpu.SemaphoreType.DMA,
    ],
)


def _dinv_of(d0_ref, d1_ref):
    deg = d0_ref[0, :, 0:1] + d1_ref[0, :, 0:1] + 1.0
    return lax.rsqrt(deg)


def _tc1_body(x_ref, w_ref, d0_ref, d1_ref, q_ref):
    dinv = _dinv_of(d0_ref, d1_ref)
    q_ref[...] = jnp.dot(x_ref[...], w_ref[...],
                         preferred_element_type=jnp.float32) * dinv


_tc1 = pl.pallas_call(
    _tc1_body,
    grid=(_GRID,),
    in_specs=[
        pl.BlockSpec((_RB, _D), lambda i: (i, 0)),
        pl.BlockSpec((_D, _H), lambda i: (0, 0)),
        pl.BlockSpec((1, _RB, _DW), lambda i: (0, i, 0)),
        pl.BlockSpec((1, _RB, _DW), lambda i: (1, i, 0)),
    ],
    out_specs=pl.BlockSpec((_RB, _H), lambda i: (i, 0)),
    out_shape=jax.ShapeDtypeStruct((_N, _H), jnp.float32),
)


def _tc2_body(s0_ref, s1_ref, q_ref, d0_ref, d1_ref, w_ref, b_ref, o_ref):
    dinv = _dinv_of(d0_ref, d1_ref)
    agg = (s0_ref[0] + s1_ref[0] + q_ref[...]) * dinv + b_ref[...]
    h = jnp.maximum(agg, 0.0)
    o_ref[...] = jnp.dot(h, w_ref[...],
                         preferred_element_type=jnp.float32) * dinv


_tc2 = pl.pallas_call(
    _tc2_body,
    grid=(_GRID,),
    in_specs=[
        pl.BlockSpec((1, _RB, _H), lambda i: (0, i, 0)),
        pl.BlockSpec((1, _RB, _H), lambda i: (1, i, 0)),
        pl.BlockSpec((_RB, _H), lambda i: (i, 0)),
        pl.BlockSpec((1, _RB, _DW), lambda i: (0, i, 0)),
        pl.BlockSpec((1, _RB, _DW), lambda i: (1, i, 0)),
        pl.BlockSpec((_H, _H), lambda i: (0, 0)),
        pl.BlockSpec((1, _H), lambda i: (0, 0)),
    ],
    out_specs=pl.BlockSpec((_RB, _H), lambda i: (i, 0)),
    out_shape=jax.ShapeDtypeStruct((_N, _H), jnp.float32),
)


def _tc3_body(s0_ref, s1_ref, q_ref, d0_ref, d1_ref, w_ref, b_ref, bo_ref,
              o_ref):
    dinv = _dinv_of(d0_ref, d1_ref)
    agg = (s0_ref[0] + s1_ref[0] + q_ref[...]) * dinv + b_ref[...]
    h = jnp.maximum(agg, 0.0)
    logits = jnp.dot(h, w_ref[...],
                     preferred_element_type=jnp.float32) + bo_ref[...]
    col = lax.broadcasted_iota(jnp.int32, (_RB, _H), 1)
    masked = jnp.where(col < _C, logits, -1e30)
    m = jnp.max(masked, axis=1, keepdims=True)
    e = jnp.exp(masked - m)
    o_ref[...] = e / jnp.sum(e, axis=1, keepdims=True)


_tc3 = pl.pallas_call(
    _tc3_body,
    grid=(_GRID,),
    in_specs=[
        pl.BlockSpec((1, _RB, _H), lambda i: (0, i, 0)),
        pl.BlockSpec((1, _RB, _H), lambda i: (1, i, 0)),
        pl.BlockSpec((_RB, _H), lambda i: (i, 0)),
        pl.BlockSpec((1, _RB, _DW), lambda i: (0, i, 0)),
        pl.BlockSpec((1, _RB, _DW), lambda i: (1, i, 0)),
        pl.BlockSpec((_H, _H), lambda i: (0, 0)),
        pl.BlockSpec((1, _H), lambda i: (0, 0)),
        pl.BlockSpec((1, _H), lambda i: (0, 0)),
    ],
    out_specs=pl.BlockSpec((_RB, _H), lambda i: (i, 0)),
    out_shape=jax.ShapeDtypeStruct((_N, _H), jnp.float32),
)


@jax.jit
def kernel(x, edge_index, W1, b1, W2, b2, W_out, b_out):
    src = edge_index[0].astype(jnp.int32)
    dst = edge_index[1].astype(jnp.int32)
    pad = _E_PAD - _E
    src_p = jnp.concatenate([src, jnp.zeros((pad,), jnp.int32)])
    dst_p = jnp.concatenate([dst, jnp.full((pad,), _N, jnp.int32)])
    dst_b = dst_p.reshape(_NW, _BLKS, _EB)
    srcu = src_p.reshape(_NW, _BLKS, _EB)
    dstu = dst_p.reshape(_NW, _BLKS, _EB)

    degp = _deg_call(dst_b)
    q1 = _tc1(x, W1, degp, degp)
    s1 = _agg_call(srcu, dstu, q1)
    q2 = _tc2(s1, s1, q1, degp, degp, W2, b1.reshape(1, _H))
    s2 = _agg_call(srcu, dstu, q2)
    w_out_p = jnp.zeros((_H, _H), jnp.float32).at[:, :_C].set(W_out)
    b_out_p = jnp.zeros((1, _H), jnp.float32).at[0, :_C].set(b_out)
    probs = _tc3(s2, s2, q2, degp, degp, w_out_p, b2.reshape(1, _H), b_out_p)
    return probs[:, :_C]


# speedup vs baseline: 1.1651x; 1.1651x over previous
import jax
import jax.numpy as jnp
from jax import lax
from jax.experimental import pallas as pl
from jax.experimental.pallas import tpu as pltpu
from jax.experimental.pallas import tpu_sc as plsc

_N = 10000
_E = 320000
_D = 128
_H = 128
_C = 7

_NC = 2
_NS = 16
_NW = _NC * _NS
_EB = 128
_BLKS = -(-_E // (_NW * _EB))
_EPW = _BLKS * _EB
_E_PAD = _NW * _EPW
_NACC = 10240
_RPT = _NACC // _NS
_DW = 16
_RB = 2000
_GRID = _N // _RB


_sc_mesh = plsc.VectorSubcoreMesh(core_axis_name="c", subcore_axis_name="s",
                                  num_cores=_NC)


def _deg_body(dst_hbm, out_hbm, ones_v, idx_v, acc):
    c = lax.axis_index("c")
    s = lax.axis_index("s")
    w = c * _NS + s

    def _fill(val):
        def row(i, carry):
            ones_v[i, :] = jnp.full((_DW,), val, jnp.float32)
            return carry
        lax.fori_loop(0, _EB, row, 0)

    _fill(0.0)
    for k in range(_RPT // _EB):
        pltpu.sync_copy(ones_v, acc.at[pl.ds(s * _RPT + k * _EB, _EB)])
    _fill(1.0)
    plsc.subcore_barrier()

    def blk(j, carry):
        off = pl.multiple_of(w * _EPW + j * _EB, 8)
        pltpu.sync_copy(dst_hbm.at[pl.ds(off, _EB)], idx_v)
        pltpu.sync_copy(ones_v, acc.at[idx_v], add=True)
        return carry

    lax.fori_loop(0, _BLKS, blk, 0)
    plsc.subcore_barrier()
    pltpu.sync_copy(acc.at[pl.ds(s * _RPT, _RPT)],
                    out_hbm.at[c, pl.ds(s * _RPT, _RPT)])


_deg_call = pl.kernel(
    _deg_body,
    out_type=jax.ShapeDtypeStruct((_NC, _NACC, _DW), jnp.float32),
    mesh=_sc_mesh,
    scratch_types=[
        pltpu.VMEM((_EB, _DW), jnp.float32),
        pltpu.VMEM((_EB,), jnp.int32),
        pltpu.VMEM_SHARED((_NACC, _DW), jnp.float32),
    ],
)


def _agg_body(src_hbm, dst_hbm, q_hbm, out_hbm, src_v, dst_v, rows_v, acc, sem):
    c = lax.axis_index("c")
    s = lax.axis_index("s")
    w = c * _NS + s

    def zrow(i, carry):
        def zcol(j, carry2):
            rows_v[i, pl.ds(pl.multiple_of(j * 16, 16), 16)] = (
                jnp.zeros((16,), jnp.float32))
            return carry2
        return lax.fori_loop(0, _H // 16, zcol, carry)

    lax.fori_loop(0, _EB, zrow, 0)
    for k in range(_RPT // _EB):
        pltpu.sync_copy(rows_v, acc.at[pl.ds(s * _RPT + k * _EB, _EB)])
    plsc.subcore_barrier()

    def blk(j, carry):
        off = pl.multiple_of(w * _EPW + j * _EB, 8)
        pltpu.sync_copy(src_hbm.at[pl.ds(off, _EB)], src_v)
        pltpu.sync_copy(dst_hbm.at[pl.ds(off, _EB)], dst_v)
        pltpu.async_copy(q_hbm.at[src_v], rows_v, sem).wait()
        pltpu.sync_copy(rows_v, acc.at[dst_v], add=True)
        return carry

    lax.fori_loop(0, _BLKS, blk, 0)
    plsc.subcore_barrier()
    pltpu.sync_copy(acc.at[pl.ds(s * _RPT, _RPT)],
                    out_hbm.at[c, pl.ds(s * _RPT, _RPT)])


_agg_call = pl.kernel(
    _agg_body,
    out_type=jax.ShapeDtypeStruct((_NC, _NACC, _H), jnp.float32),
    mesh=_sc_mesh,
    scratch_types=[
        pltpu.VMEM((_EB,), jnp.int32),
        pltpu.VMEM((_EB,), jnp.int32),
        pltpu.VMEM((_EB, _H), jnp.float32),
        pltpu.VMEM_SHARED((_NACC, _H), jnp.float32),
        pltpu.SemaphoreType.DMA,
    ],
)


def _dinv_of(d0_ref, d1_ref):
    deg = d0_ref[0, :, 0:1] + d1_ref[0, :, 0:1] + 1.0
    return lax.rsqrt(deg)


def _tc1_body(x_ref, w_ref, d0_ref, d1_ref, q_ref):
    dinv = _dinv_of(d0_ref, d1_ref)
    q_ref[...] = jnp.dot(x_ref[...], w_ref[...],
                         preferred_element_type=jnp.float32) * dinv


_tc1 = pl.pallas_call(
    _tc1_body,
    grid=(_GRID,),
    in_specs=[
        pl.BlockSpec((_RB, _D), lambda i: (i, 0)),
        pl.BlockSpec((_D, _H), lambda i: (0, 0)),
        pl.BlockSpec((1, _RB, _DW), lambda i: (0, i, 0)),
        pl.BlockSpec((1, _RB, _DW), lambda i: (1, i, 0)),
    ],
    out_specs=pl.BlockSpec((_RB, _H), lambda i: (i, 0)),
    out_shape=jax.ShapeDtypeStruct((_N, _H), jnp.float32),
)


def _tc2_body(s0_ref, s1_ref, q_ref, d0_ref, d1_ref, w_ref, b_ref, o_ref):
    dinv = _dinv_of(d0_ref, d1_ref)
    agg = (s0_ref[0] + s1_ref[0] + q_ref[...]) * dinv + b_ref[...]
    h = jnp.maximum(agg, 0.0)
    o_ref[...] = jnp.dot(h, w_ref[...],
                         preferred_element_type=jnp.float32) * dinv


_tc2 = pl.pallas_call(
    _tc2_body,
    grid=(_GRID,),
    in_specs=[
        pl.BlockSpec((1, _RB, _H), lambda i: (0, i, 0)),
        pl.BlockSpec((1, _RB, _H), lambda i: (1, i, 0)),
        pl.BlockSpec((_RB, _H), lambda i: (i, 0)),
        pl.BlockSpec((1, _RB, _DW), lambda i: (0, i, 0)),
        pl.BlockSpec((1, _RB, _DW), lambda i: (1, i, 0)),
        pl.BlockSpec((_H, _H), lambda i: (0, 0)),
        pl.BlockSpec((1, _H), lambda i: (0, 0)),
    ],
    out_specs=pl.BlockSpec((_RB, _H), lambda i: (i, 0)),
    out_shape=jax.ShapeDtypeStruct((_N, _H), jnp.float32),
)


def _tc3_body(s0_ref, s1_ref, q_ref, d0_ref, d1_ref, w_ref, b_ref, bo_ref,
              o_ref):
    dinv = _dinv_of(d0_ref, d1_ref)
    agg = (s0_ref[0] + s1_ref[0] + q_ref[...]) * dinv + b_ref[...]
    h = jnp.maximum(agg, 0.0)
    logits = jnp.dot(h, w_ref[...],
                     preferred_element_type=jnp.float32) + bo_ref[...]
    col = lax.broadcasted_iota(jnp.int32, (_RB, _H), 1)
    masked = jnp.where(col < _C, logits, -1e30)
    m = jnp.max(masked, axis=1, keepdims=True)
    e = jnp.exp(masked - m)
    o_ref[...] = e / jnp.sum(e, axis=1, keepdims=True)


_tc3 = pl.pallas_call(
    _tc3_body,
    grid=(_GRID,),
    in_specs=[
        pl.BlockSpec((1, _RB, _H), lambda i: (0, i, 0)),
        pl.BlockSpec((1, _RB, _H), lambda i: (1, i, 0)),
        pl.BlockSpec((_RB, _H), lambda i: (i, 0)),
        pl.BlockSpec((1, _RB, _DW), lambda i: (0, i, 0)),
        pl.BlockSpec((1, _RB, _DW), lambda i: (1, i, 0)),
        pl.BlockSpec((_H, _H), lambda i: (0, 0)),
        pl.BlockSpec((1, _H), lambda i: (0, 0)),
        pl.BlockSpec((1, _H), lambda i: (0, 0)),
    ],
    out_specs=pl.BlockSpec((_RB, _H), lambda i: (i, 0)),
    out_shape=jax.ShapeDtypeStruct((_N, _H), jnp.float32),
)


@jax.jit
def kernel(x, edge_index, W1, b1, W2, b2, W_out, b_out):
    src = edge_index[0].astype(jnp.int32)
    dst = edge_index[1].astype(jnp.int32)
    pad = _E_PAD - _E
    src_p = jnp.concatenate([src, jnp.zeros((pad,), jnp.int32)])
    dst_p = jnp.concatenate([dst, jnp.full((pad,), _N, jnp.int32)])

    degp = _deg_call(dst_p)
    q1 = _tc1(x, W1, degp, degp)
    s1 = _agg_call(src_p, dst_p, q1)
    q2 = _tc2(s1, s1, q1, degp, degp, W2, b1.reshape(1, _H))
    s2 = _agg_call(src_p, dst_p, q2)
    w_out_p = jnp.zeros((_H, _H), jnp.float32).at[:, :_C].set(W_out)
    b_out_p = jnp.zeros((1, _H), jnp.float32).at[0, :_C].set(b_out)
    probs = _tc3(s2, s2, q2, degp, degp, w_out_p, b2.reshape(1, _H), b_out_p)
    return probs[:, :_C]
